# Initial kernel scaffold; baseline (speedup 1.0000x reference)
import jax
import jax.numpy as jnp
from jax.experimental import pallas as pl


def kernel(x, conv_w, conv_b, Wq, bq, Wk, bk, Wv, bv, Wo, bo, ln1_g, ln1_b, ffn1_w, ffn1_b, ffn2_w, ffn2_b, ln2_g, ln2_b, lnf_g, lnf_b):
    raise NotImplementedError("write your pallas kernel here")


# speedup vs baseline: 3.9773x; 3.9773x over previous
import functools
import math

import numpy as np
import jax
import jax.numpy as jnp
from jax import lax
from jax.experimental import pallas as pl
from jax.experimental.pallas import tpu as pltpu

B, L, ENC_IN = 2, 2048, 7
D_MODEL, N_HEADS, E_LAYERS, D_FF = 768, 12, 2, 3072
FACTOR = 5
DH = D_MODEL // N_HEADS
U = int(min(FACTOR * math.ceil(math.log(L)), L))
SAMPLE_K = int(min(FACTOR * math.ceil(math.log(L)), L))
ROWS = B * L
NEG = -3e38


def _pos_encoding():
    position = np.arange(L, dtype=np.float32)[:, None]
    div_term = np.exp(
        np.arange(0, D_MODEL, 2, dtype=np.float32) * (-math.log(10000.0) / D_MODEL))
    pe = np.zeros((L, D_MODEL), dtype=np.float32)
    pe[:, 0::2] = np.sin(position * div_term)
    pe[:, 1::2] = np.cos(position * div_term)
    return pe


def _threefry2x32(k0, k1, x0, x1):
    def rotl(x, d):
        return ((x << np.uint32(d)) | (x >> np.uint32(32 - d))).astype(np.uint32)
    ks0 = np.uint32(k0)
    ks1 = np.uint32(k1)
    ks2 = np.uint32(ks0 ^ ks1 ^ np.uint32(0x1BD11BDA))
    ks = [ks0, ks1, ks2]
    rot = [[13, 15, 26, 6], [17, 29, 16, 24]]
    x0 = (np.asarray(x0, np.uint32) + ks0).astype(np.uint32)
    x1 = (np.asarray(x1, np.uint32) + ks1).astype(np.uint32)
    for i in range(5):
        for r in rot[i % 2]:
            x0 = (x0 + x1).astype(np.uint32)
            x1 = rotl(x1, r)
            x1 = (x1 ^ x0).astype(np.uint32)
        x0 = (x0 + ks[(i + 1) % 3]).astype(np.uint32)
        x1 = (x1 + ks[(i + 2) % 3] + np.uint32(i + 1)).astype(np.uint32)
    return x0, x1


def _np_bits(k0, k1, size):
    cnt = np.arange(size, dtype=np.uint64)
    hi = (cnt >> np.uint64(32)).astype(np.uint32)
    lo = (cnt & np.uint64(0xFFFFFFFF)).astype(np.uint32)
    o0, o1 = _threefry2x32(k0, k1, hi, lo)
    return o0 ^ o1


@functools.lru_cache(maxsize=None)
def _sample_counts():
    outs = []
    for i in range(E_LAYERS):
        f0, f1 = _threefry2x32(0, 42, np.uint32(0), np.uint32(i))
        s0, s1 = _threefry2x32(int(f0), int(f1),
                               np.zeros(2, np.uint32),
                               np.arange(2, dtype=np.uint32))
        lower = _np_bits(int(s0[1]), int(s1[1]), L * SAMPLE_K)
        idx = (lower % np.uint32(L)).astype(np.int32).reshape(L, SAMPLE_K)
        cnt = np.zeros((L, L), dtype=np.int8)
        np.add.at(cnt, (np.arange(L)[:, None], idx), 1)
        outs.append(np.ascontiguousarray(cnt.T))
    return outs


_PE = _pos_encoding()
_CNTS = _sample_counts()


def _embed_kernel(x_ref, w_ref, b_ref, pe_ref, o_ref):
    acc = jnp.dot(x_ref[...], w_ref[...], preferred_element_type=jnp.float32)
    o_ref[...] = acc + b_ref[...] + pe_ref[...]


def _embed(xcat, w2d, bias, pe):
    blk = 512
    grid = (ROWS // blk,)
    return pl.pallas_call(
        _embed_kernel,
        grid=grid,
        in_specs=[
            pl.BlockSpec((blk, ENC_IN * 3), lambda i: (i, 0)),
            pl.BlockSpec((ENC_IN * 3, D_MODEL), lambda i: (0, 0)),
            pl.BlockSpec((1, D_MODEL), lambda i: (0, 0)),
            pl.BlockSpec((blk, D_MODEL), lambda i: (i % (L // blk), 0)),
        ],
        out_specs=pl.BlockSpec((blk, D_MODEL), lambda i: (i, 0)),
        out_shape=jax.ShapeDtypeStruct((ROWS, D_MODEL), jnp.float32),
    )(xcat, w2d, bias, pe)


def _matmul_bias_kernel(a_ref, w_ref, b_ref, o_ref):
    o_ref[...] = (
        jnp.dot(a_ref[...], w_ref[...], preferred_element_type=jnp.float32)
        + b_ref[...])


def _matmul_bias(a, w, b):
    blk = 512
    n = w.shape[1]
    grid = (ROWS // blk,)
    return pl.pallas_call(
        _matmul_bias_kernel,
        grid=grid,
        in_specs=[
            pl.BlockSpec((blk, D_MODEL), lambda i: (i, 0)),
            pl.BlockSpec((D_MODEL, n), lambda i: (0, 0)),
            pl.BlockSpec((1, n), lambda i: (0, 0)),
        ],
        out_specs=pl.BlockSpec((blk, n), lambda i: (i, 0)),
        out_shape=jax.ShapeDtypeStruct((ROWS, n), jnp.float32),
    )(a, w, b)


def _attn_kernel(q_ref, k_ref, v_ref, c_ref, o_ref, oh_ref):
    q = q_ref[0, 0]
    k = k_ref[0, 0]
    v = v_ref[0, 0]

    chunk = 512
    m_parts = []
    for ci in range(L // chunk):
        qc = q[ci * chunk:(ci + 1) * chunk]
        st = lax.dot_general(k, qc, (((1,), (1,)), ((), ())),
                             preferred_element_type=jnp.float32)
        cT = c_ref[:, ci * chunk:(ci + 1) * chunk]
        cf = cT.astype(jnp.float32)
        smax = jnp.max(jnp.where(cf > 0.0, st, NEG), axis=0, keepdims=True)
        ssum = jnp.sum(st * cf, axis=0, keepdims=True)
        m_parts.append(smax - ssum * (1.0 / L))
    m = jnp.concatenate(m_parts, axis=1)

    iota = lax.broadcasted_iota(jnp.int32, (1, L), 1)
    for i in range(U):
        mx = jnp.max(m, axis=1, keepdims=True)
        elig = m >= mx
        a = jnp.min(jnp.where(elig, iota, L), axis=1, keepdims=True)
        row = iota == a
        oh_ref[i:i + 1, :] = row.astype(jnp.float32)
        m = jnp.where(row, NEG, m)

    oh = oh_ref[...]
    qr = jnp.dot(oh, q, preferred_element_type=jnp.float32)
    scores = lax.dot_general(qr, k, (((1,), (1,)), ((), ())),
                             preferred_element_type=jnp.float32)
    scores = scores * (1.0 / math.sqrt(DH))
    smax = jnp.max(scores, axis=1, keepdims=True)
    e = jnp.exp(scores - smax)
    attnw = e / jnp.sum(e, axis=1, keepdims=True)
    upd = jnp.dot(attnw, v, preferred_element_type=jnp.float32)

    meanv = jnp.sum(v, axis=0, keepdims=True) * (1.0 / L)
    scat = lax.dot_general(oh, upd, (((0,), (0,)), ((), ())),
                           preferred_element_type=jnp.float32)
    sel = lax.dot_general(oh, jnp.ones((U, DH), jnp.float32),
                          (((0,), (0,)), ((), ())),
                          preferred_element_type=jnp.float32)
    o_ref[0, 0] = scat + meanv * (1.0 - sel)


def _attention(q, k, v, cntT):
    grid = (B, N_HEADS)
    spec = pl.BlockSpec((1, 1, L, DH), lambda b, h: (b, h, 0, 0))
    return pl.pallas_call(
        _attn_kernel,
        grid=grid,
        in_specs=[
            spec, spec, spec,
            pl.BlockSpec((L, L), lambda b, h: (0, 0)),
        ],
        out_specs=spec,
        out_shape=jax.ShapeDtypeStruct((B, N_HEADS, L, DH), jnp.float32),
        scratch_shapes=[pltpu.VMEM((U, L), jnp.float32)],
    )(q, k, v, cntT)


def _ln(x, g, b):
    mu = jnp.mean(x, axis=1, keepdims=True)
    xc = x - mu
    var = jnp.mean(xc * xc, axis=1, keepdims=True)
    return xc * lax.rsqrt(var + 1e-5) * g + b


def _proj_ln_kernel(a_ref, w_ref, b_ref, h_ref, g_ref, gb_ref, o_ref):
    out = (jnp.dot(a_ref[...], w_ref[...], preferred_element_type=jnp.float32)
           + b_ref[...])
    o_ref[...] = _ln(h_ref[...] + out, g_ref[...], gb_ref[...])


def _proj_ln(a, w, b, h, g, gb):
    blk = 512
    grid = (ROWS // blk,)
    return pl.pallas_call(
        _proj_ln_kernel,
        grid=grid,
        in_specs=[
            pl.BlockSpec((blk, D_MODEL), lambda i: (i, 0)),
            pl.BlockSpec((D_MODEL, D_MODEL), lambda i: (0, 0)),
            pl.BlockSpec((1, D_MODEL), lambda i: (0, 0)),
            pl.BlockSpec((blk, D_MODEL), lambda i: (i, 0)),
            pl.BlockSpec((1, D_MODEL), lambda i: (0, 0)),
            pl.BlockSpec((1, D_MODEL), lambda i: (0, 0)),
        ],
        out_specs=pl.BlockSpec((blk, D_MODEL), lambda i: (i, 0)),
        out_shape=jax.ShapeDtypeStruct((ROWS, D_MODEL), jnp.float32),
    )(a, w, b, h, g, gb)


def _ffn_kernel(h_ref, w1_ref, b1_ref, w2_ref, b2_ref, g_ref, gb_ref,
                fg_ref, fb_ref, o_ref, *, final_ln):
    h = h_ref[...]
    y = jnp.maximum(
        jnp.dot(h, w1_ref[...], preferred_element_type=jnp.float32)
        + b1_ref[...], 0.0)
    y = (jnp.dot(y, w2_ref[...], preferred_element_type=jnp.float32)
         + b2_ref[...])
    out = _ln(h + y, g_ref[...], gb_ref[...])
    if final_ln:
        out = _ln(out, fg_ref[...], fb_ref[...])
    o_ref[...] = out


def _ffn(h, w1, b1, w2, b2, g, gb, fg, fb, final_ln):
    blk = 256
    grid = (ROWS // blk,)
    vec = lambda n: pl.BlockSpec((1, n), lambda i: (0, 0))
    return pl.pallas_call(
        functools.partial(_ffn_kernel, final_ln=final_ln),
        grid=grid,
        in_specs=[
            pl.BlockSpec((blk, D_MODEL), lambda i: (i, 0)),
            pl.BlockSpec((D_MODEL, D_FF), lambda i: (0, 0)),
            vec(D_FF),
            pl.BlockSpec((D_FF, D_MODEL), lambda i: (0, 0)),
            vec(D_MODEL),
            vec(D_MODEL),
            vec(D_MODEL),
            vec(D_MODEL),
            vec(D_MODEL),
        ],
        out_specs=pl.BlockSpec((blk, D_MODEL), lambda i: (i, 0)),
        out_shape=jax.ShapeDtypeStruct((ROWS, D_MODEL), jnp.float32),
    )(h, w1, b1, w2, b2, g, gb, fg, fb)


def kernel(x, conv_w, conv_b, Wq, bq, Wk, bk, Wv, bv, Wo, bo, ln1_g, ln1_b,
           ffn1_w, ffn1_b, ffn2_w, ffn2_b, ln2_g, ln2_b, lnf_g, lnf_b):
    cnts = _CNTS
    pe = jnp.asarray(_PE)

    xprev = jnp.roll(x, 1, axis=1)
    xnext = jnp.roll(x, -1, axis=1)
    xcat = jnp.concatenate([xprev, x, xnext], axis=-1).reshape(ROWS, 3 * ENC_IN)
    w2d = conv_w.transpose(2, 1, 0).reshape(3 * ENC_IN, D_MODEL)
    h = _embed(xcat, w2d, conv_b.reshape(1, D_MODEL), pe)

    for i in range(E_LAYERS):
        wcat = jnp.concatenate([Wq[i].T, Wk[i].T, Wv[i].T], axis=1)
        bcat = jnp.concatenate([bq[i], bk[i], bv[i]]).reshape(1, 3 * D_MODEL)
        qkv = _matmul_bias(h, wcat, bcat)
        qkv4 = qkv.reshape(B, L, 3, N_HEADS, DH).transpose(2, 0, 3, 1, 4)
        ctx = _attention(qkv4[0], qkv4[1], qkv4[2], jnp.asarray(cnts[i]))
        ctx2 = ctx.transpose(0, 2, 1, 3).reshape(ROWS, D_MODEL)
        h = _proj_ln(ctx2, Wo[i].T, bo[i].reshape(1, D_MODEL), h,
                     ln1_g[i].reshape(1, D_MODEL), ln1_b[i].reshape(1, D_MODEL))
        h = _ffn(h, ffn1_w[i].T, ffn1_b[i].reshape(1, D_FF), ffn2_w[i].T,
                 ffn2_b[i].reshape(1, D_MODEL),
                 ln2_g[i].reshape(1, D_MODEL), ln2_b[i].reshape(1, D_MODEL),
                 lnf_g.reshape(1, D_MODEL), lnf_b.reshape(1, D_MODEL),
                 final_ln=(i == E_LAYERS - 1))

    return h.reshape(B, L, D_MODEL)


# speedup vs baseline: 7.5665x; 1.9024x over previous
import functools
import math

import numpy as np
import jax
import jax.numpy as jnp
from jax import lax
from jax.experimental import pallas as pl
from jax.experimental.pallas import tpu as pltpu

B, L, ENC_IN = 2, 2048, 7
D_MODEL, N_HEADS, E_LAYERS, D_FF = 768, 12, 2, 3072
FACTOR = 5
DH = D_MODEL // N_HEADS
U = int(min(FACTOR * math.ceil(math.log(L)), L))
SAMPLE_K = int(min(FACTOR * math.ceil(math.log(L)), L))
ROWS = B * L
NEG = -3e38


def _pos_encoding():
    position = np.arange(L, dtype=np.float32)[:, None]
    div_term = np.exp(
        np.arange(0, D_MODEL, 2, dtype=np.float32) * (-math.log(10000.0) / D_MODEL))
    pe = np.zeros((L, D_MODEL), dtype=np.float32)
    pe[:, 0::2] = np.sin(position * div_term)
    pe[:, 1::2] = np.cos(position * div_term)
    return pe


def _threefry2x32(k0, k1, x0, x1):
    def rotl(x, d):
        return ((x << np.uint32(d)) | (x >> np.uint32(32 - d))).astype(np.uint32)
    ks0 = np.uint32(k0)
    ks1 = np.uint32(k1)
    ks2 = np.uint32(ks0 ^ ks1 ^ np.uint32(0x1BD11BDA))
    ks = [ks0, ks1, ks2]
    rot = [[13, 15, 26, 6], [17, 29, 16, 24]]
    x0 = (np.asarray(x0, np.uint32) + ks0).astype(np.uint32)
    x1 = (np.asarray(x1, np.uint32) + ks1).astype(np.uint32)
    for i in range(5):
        for r in rot[i % 2]:
            x0 = (x0 + x1).astype(np.uint32)
            x1 = rotl(x1, r)
            x1 = (x1 ^ x0).astype(np.uint32)
        x0 = (x0 + ks[(i + 1) % 3]).astype(np.uint32)
        x1 = (x1 + ks[(i + 2) % 3] + np.uint32(i + 1)).astype(np.uint32)
    return x0, x1


def _np_bits(k0, k1, size):
    cnt = np.arange(size, dtype=np.uint64)
    hi = (cnt >> np.uint64(32)).astype(np.uint32)
    lo = (cnt & np.uint64(0xFFFFFFFF)).astype(np.uint32)
    o0, o1 = _threefry2x32(k0, k1, hi, lo)
    return o0 ^ o1


@functools.lru_cache(maxsize=None)
def _sample_counts():
    outs = []
    for i in range(E_LAYERS):
        f0, f1 = _threefry2x32(0, 42, np.uint32(0), np.uint32(i))
        s0, s1 = _threefry2x32(int(f0), int(f1),
                               np.zeros(2, np.uint32),
                               np.arange(2, dtype=np.uint32))
        lower = _np_bits(int(s0[1]), int(s1[1]), L * SAMPLE_K)
        idx = (lower % np.uint32(L)).astype(np.int32).reshape(L, SAMPLE_K)
        cnt = np.zeros((L, L), dtype=np.float32)
        np.add.at(cnt, (np.arange(L)[:, None], idx), 1.0)
        cfT = np.ascontiguousarray(cnt.T)
        nmT = np.where(cfT > 0, np.float32(0.0), np.float32(NEG)).astype(np.float32)
        outs.append((nmT, cfT))
    return outs


_PE = _pos_encoding()
_CNTS = _sample_counts()


def _embed_kernel(x_ref, w_ref, b_ref, pe_ref, o_ref):
    acc = jnp.dot(x_ref[...], w_ref[...], preferred_element_type=jnp.float32)
    o_ref[...] = acc + b_ref[...] + pe_ref[...]


def _embed(xcat, w2d, bias, pe):
    blk = 512
    grid = (ROWS // blk,)
    return pl.pallas_call(
        _embed_kernel,
        grid=grid,
        in_specs=[
            pl.BlockSpec((blk, ENC_IN * 3), lambda i: (i, 0)),
            pl.BlockSpec((ENC_IN * 3, D_MODEL), lambda i: (0, 0)),
            pl.BlockSpec((1, D_MODEL), lambda i: (0, 0)),
            pl.BlockSpec((blk, D_MODEL), lambda i: (i % (L // blk), 0)),
        ],
        out_specs=pl.BlockSpec((blk, D_MODEL), lambda i: (i, 0)),
        out_shape=jax.ShapeDtypeStruct((ROWS, D_MODEL), jnp.float32),
    )(xcat, w2d, bias, pe)


def _matmul_bias_kernel(a_ref, w_ref, b_ref, o_ref):
    o_ref[...] = (
        jnp.dot(a_ref[...], w_ref[...], preferred_element_type=jnp.float32)
        + b_ref[...])


def _matmul_bias(a, w, b):
    blk = 512
    n = w.shape[1]
    grid = (ROWS // blk,)
    return pl.pallas_call(
        _matmul_bias_kernel,
        grid=grid,
        in_specs=[
            pl.BlockSpec((blk, D_MODEL), lambda i: (i, 0)),
            pl.BlockSpec((D_MODEL, n), lambda i: (0, 0)),
            pl.BlockSpec((1, n), lambda i: (0, 0)),
        ],
        out_specs=pl.BlockSpec((blk, n), lambda i: (i, 0)),
        out_shape=jax.ShapeDtypeStruct((ROWS, n), jnp.float32),
    )(a, w, b)


def _attn_head_body(q, k, v, nm_ref, cf_ref):
    ckT = lax.dot_general(k, cf_ref[...], (((0,), (0,)), ((), ())),
                          preferred_element_type=jnp.float32)
    qT = q.T
    ssum = jnp.sum(qT * ckT, axis=0, keepdims=True)

    chunk = 512
    m_parts = []
    for ci in range(L // chunk):
        qc = q[ci * chunk:(ci + 1) * chunk]
        st = lax.dot_general(k, qc, (((1,), (1,)), ((), ())),
                             preferred_element_type=jnp.float32)
        w = st + nm_ref[:, ci * chunk:(ci + 1) * chunk]
        m_parts.append(jnp.max(w, axis=0, keepdims=True))
    m = jnp.concatenate(m_parts, axis=1) - ssum * (1.0 / L)

    ub = lax.bitcast_convert_type(m, jnp.int32)
    si = ub ^ (lax.shift_right_arithmetic(ub, 31) & jnp.int32(0x7FFFFFFF))

    def count_ge(t):
        return jnp.sum(jnp.where(si >= t, 1.0, 0.0), axis=1, keepdims=True)

    c0 = count_ge(jnp.zeros((1, 1), jnp.int32))
    t = jnp.where(c0 >= U, jnp.int32(0), jnp.int32(-2147483648)
                  ).reshape(1, 1).astype(jnp.int32)
    for b in range(30, -1, -1):
        t_try = t + jnp.int32(1 << b)
        t = jnp.where(count_ge(t_try) >= U, t_try, t)

    mask_gt = (si > t).astype(jnp.float32)
    mask_eq = (si == t).astype(jnp.float32)
    r = U - jnp.sum(mask_gt, axis=1, keepdims=True)
    lt128 = jnp.where(
        lax.broadcasted_iota(jnp.int32, (128, 128), 0)
        <= lax.broadcasted_iota(jnp.int32, (128, 128), 1),
        1.0, 0.0).astype(jnp.bfloat16)
    lt16x = jnp.where(
        lax.broadcasted_iota(jnp.int32, (16, 16), 1)
        < lax.broadcasted_iota(jnp.int32, (16, 16), 0),
        1.0, 0.0)

    def prefix(x_row):
        xb = x_row.reshape(16, 128)
        pb = jnp.dot(xb.astype(jnp.bfloat16), lt128,
                     preferred_element_type=jnp.float32)
        off = jnp.dot(lt16x, pb[:, 127:128],
                      preferred_element_type=jnp.float32)
        return (pb + off).reshape(1, L)

    pos_eq = prefix(mask_eq)
    mask = mask_gt + mask_eq * jnp.where(pos_eq <= r, 1.0, 0.0)
    pos = prefix(mask)

    rows = lax.broadcasted_iota(jnp.int32, (U, L), 0).astype(jnp.float32) + 1.0
    oh = jnp.where((pos == rows) & (mask > 0.0), 1.0, 0.0)
    qr = jnp.dot(oh, q, preferred_element_type=jnp.float32)
    scores = lax.dot_general(qr, k, (((1,), (1,)), ((), ())),
                             preferred_element_type=jnp.float32)
    scores = scores * (1.0 / math.sqrt(DH))
    smax = jnp.max(scores, axis=1, keepdims=True)
    e = jnp.exp(scores - smax)
    attnw = e / jnp.sum(e, axis=1, keepdims=True)
    upd = jnp.dot(attnw, v, preferred_element_type=jnp.float32)

    meanv = jnp.sum(v, axis=0, keepdims=True) * (1.0 / L)
    scat = lax.dot_general(oh, upd, (((0,), (0,)), ((), ())),
                           preferred_element_type=jnp.float32)
    sel = lax.dot_general(oh, jnp.ones((U, DH), jnp.float32),
                          (((0,), (0,)), ((), ())),
                          preferred_element_type=jnp.float32)
    return scat + meanv * (1.0 - sel)


def _attn_kernel(qp_ref, kp_ref, vp_ref, nm_ref, cf_ref, o_ref):
    qp = qp_ref[0]
    kp = kp_ref[0]
    vp = vp_ref[0]
    ctxs = []
    for hh in range(2):
        q = qp[:, hh * DH:(hh + 1) * DH]
        k = kp[:, hh * DH:(hh + 1) * DH]
        v = vp[:, hh * DH:(hh + 1) * DH]
        ctxs.append(_attn_head_body(q, k, v, nm_ref, cf_ref))
    o_ref[0] = jnp.concatenate(ctxs, axis=1)


def _attention(qkv3, nmT, cfT):
    npair = N_HEADS // 2
    grid = (B, npair)
    qs = pl.BlockSpec((1, L, 2 * DH), lambda b, p: (b, 0, p))
    ks = pl.BlockSpec((1, L, 2 * DH), lambda b, p: (b, 0, npair + p))
    vs = pl.BlockSpec((1, L, 2 * DH), lambda b, p: (b, 0, 2 * npair + p))
    full = pl.BlockSpec((L, L), lambda b, p: (0, 0))
    return pl.pallas_call(
        _attn_kernel,
        grid=grid,
        in_specs=[qs, ks, vs, full, full],
        out_specs=pl.BlockSpec((1, L, 2 * DH), lambda b, p: (b, 0, p)),
        out_shape=jax.ShapeDtypeStruct((B, L, D_MODEL), jnp.float32),
    )(qkv3, qkv3, qkv3, nmT, cfT)


def _ln(x, g, b):
    mu = jnp.mean(x, axis=1, keepdims=True)
    xc = x - mu
    var = jnp.mean(xc * xc, axis=1, keepdims=True)
    return xc * lax.rsqrt(var + 1e-5) * g + b


def _proj_ln_kernel(a_ref, w_ref, b_ref, h_ref, g_ref, gb_ref, o_ref):
    out = (jnp.dot(a_ref[...], w_ref[...], preferred_element_type=jnp.float32)
           + b_ref[...])
    o_ref[...] = _ln(h_ref[...] + out, g_ref[...], gb_ref[...])


def _proj_ln(a, w, b, h, g, gb):
    blk = 512
    grid = (ROWS // blk,)
    return pl.pallas_call(
        _proj_ln_kernel,
        grid=grid,
        in_specs=[
            pl.BlockSpec((blk, D_MODEL), lambda i: (i, 0)),
            pl.BlockSpec((D_MODEL, D_MODEL), lambda i: (0, 0)),
            pl.BlockSpec((1, D_MODEL), lambda i: (0, 0)),
            pl.BlockSpec((blk, D_MODEL), lambda i: (i, 0)),
            pl.BlockSpec((1, D_MODEL), lambda i: (0, 0)),
            pl.BlockSpec((1, D_MODEL), lambda i: (0, 0)),
        ],
        out_specs=pl.BlockSpec((blk, D_MODEL), lambda i: (i, 0)),
        out_shape=jax.ShapeDtypeStruct((ROWS, D_MODEL), jnp.float32),
    )(a, w, b, h, g, gb)


def _ffn_kernel(h_ref, w1_ref, b1_ref, w2_ref, b2_ref, g_ref, gb_ref,
                fg_ref, fb_ref, o_ref, *, final_ln):
    h = h_ref[...]
    y = jnp.maximum(
        jnp.dot(h, w1_ref[...], preferred_element_type=jnp.float32)
        + b1_ref[...], 0.0)
    y = (jnp.dot(y, w2_ref[...], preferred_element_type=jnp.float32)
         + b2_ref[...])
    out = _ln(h + y, g_ref[...], gb_ref[...])
    if final_ln:
        out = _ln(out, fg_ref[...], fb_ref[...])
    o_ref[...] = out


def _ffn(h, w1, b1, w2, b2, g, gb, fg, fb, final_ln):
    blk = 256
    grid = (ROWS // blk,)
    vec = lambda n: pl.BlockSpec((1, n), lambda i: (0, 0))
    return pl.pallas_call(
        functools.partial(_ffn_kernel, final_ln=final_ln),
        grid=grid,
        in_specs=[
            pl.BlockSpec((blk, D_MODEL), lambda i: (i, 0)),
            pl.BlockSpec((D_MODEL, D_FF), lambda i: (0, 0)),
            vec(D_FF),
            pl.BlockSpec((D_FF, D_MODEL), lambda i: (0, 0)),
            vec(D_MODEL),
            vec(D_MODEL),
            vec(D_MODEL),
            vec(D_MODEL),
            vec(D_MODEL),
        ],
        out_specs=pl.BlockSpec((blk, D_MODEL), lambda i: (i, 0)),
        out_shape=jax.ShapeDtypeStruct((ROWS, D_MODEL), jnp.float32),
    )(h, w1, b1, w2, b2, g, gb, fg, fb)


def kernel(x, conv_w, conv_b, Wq, bq, Wk, bk, Wv, bv, Wo, bo, ln1_g, ln1_b,
           ffn1_w, ffn1_b, ffn2_w, ffn2_b, ln2_g, ln2_b, lnf_g, lnf_b):
    cnts = _CNTS
    pe = jnp.asarray(_PE)

    xprev = jnp.roll(x, 1, axis=1)
    xnext = jnp.roll(x, -1, axis=1)
    xcat = jnp.concatenate([xprev, x, xnext], axis=-1).reshape(ROWS, 3 * ENC_IN)
    w2d = conv_w.transpose(2, 1, 0).reshape(3 * ENC_IN, D_MODEL)
    h = _embed(xcat, w2d, conv_b.reshape(1, D_MODEL), pe)

    for i in range(E_LAYERS):
        wcat = jnp.concatenate([Wq[i].T, Wk[i].T, Wv[i].T], axis=1)
        bcat = jnp.concatenate([bq[i], bk[i], bv[i]]).reshape(1, 3 * D_MODEL)
        qkv = _matmul_bias(h, wcat, bcat)
        ctx = _attention(qkv.reshape(B, L, 3 * D_MODEL),
                         jnp.asarray(cnts[i][0]), jnp.asarray(cnts[i][1]))
        ctx2 = ctx.reshape(ROWS, D_MODEL)
        h = _proj_ln(ctx2, Wo[i].T, bo[i].reshape(1, D_MODEL), h,
                     ln1_g[i].reshape(1, D_MODEL), ln1_b[i].reshape(1, D_MODEL))
        h = _ffn(h, ffn1_w[i].T, ffn1_b[i].reshape(1, D_FF), ffn2_w[i].T,
                 ffn2_b[i].reshape(1, D_MODEL),
                 ln2_g[i].reshape(1, D_MODEL), ln2_b[i].reshape(1, D_MODEL),
                 lnf_g.reshape(1, D_MODEL), lnf_b.reshape(1, D_MODEL),
                 final_ln=(i == E_LAYERS - 1))

    return h.reshape(B, L, D_MODEL)


# speedup vs baseline: 8.7223x; 1.1528x over previous
import functools
import math

import numpy as np
import jax
import jax.numpy as jnp
from jax import lax
from jax.experimental import pallas as pl
from jax.experimental.pallas import tpu as pltpu

B, L, ENC_IN = 2, 2048, 7
D_MODEL, N_HEADS, E_LAYERS, D_FF = 768, 12, 2, 3072
FACTOR = 5
DH = D_MODEL // N_HEADS
U = int(min(FACTOR * math.ceil(math.log(L)), L))
SAMPLE_K = int(min(FACTOR * math.ceil(math.log(L)), L))
ROWS = B * L
NEG = -3e38


def _pos_encoding():
    position = np.arange(L, dtype=np.float32)[:, None]
    div_term = np.exp(
        np.arange(0, D_MODEL, 2, dtype=np.float32) * (-math.log(10000.0) / D_MODEL))
    pe = np.zeros((L, D_MODEL), dtype=np.float32)
    pe[:, 0::2] = np.sin(position * div_term)
    pe[:, 1::2] = np.cos(position * div_term)
    return pe


def _threefry2x32(k0, k1, x0, x1):
    def rotl(x, d):
        return ((x << np.uint32(d)) | (x >> np.uint32(32 - d))).astype(np.uint32)
    ks0 = np.uint32(k0)
    ks1 = np.uint32(k1)
    ks2 = np.uint32(ks0 ^ ks1 ^ np.uint32(0x1BD11BDA))
    ks = [ks0, ks1, ks2]
    rot = [[13, 15, 26, 6], [17, 29, 16, 24]]
    x0 = (np.asarray(x0, np.uint32) + ks0).astype(np.uint32)
    x1 = (np.asarray(x1, np.uint32) + ks1).astype(np.uint32)
    for i in range(5):
        for r in rot[i % 2]:
            x0 = (x0 + x1).astype(np.uint32)
            x1 = rotl(x1, r)
            x1 = (x1 ^ x0).astype(np.uint32)
        x0 = (x0 + ks[(i + 1) % 3]).astype(np.uint32)
        x1 = (x1 + ks[(i + 2) % 3] + np.uint32(i + 1)).astype(np.uint32)
    return x0, x1


def _np_bits(k0, k1, size):
    cnt = np.arange(size, dtype=np.uint64)
    hi = (cnt >> np.uint64(32)).astype(np.uint32)
    lo = (cnt & np.uint64(0xFFFFFFFF)).astype(np.uint32)
    o0, o1 = _threefry2x32(k0, k1, hi, lo)
    return o0 ^ o1


@functools.lru_cache(maxsize=None)
def _sample_counts():
    outs = []
    for i in range(E_LAYERS):
        f0, f1 = _threefry2x32(0, 42, np.uint32(0), np.uint32(i))
        s0, s1 = _threefry2x32(int(f0), int(f1),
                               np.zeros(2, np.uint32),
                               np.arange(2, dtype=np.uint32))
        lower = _np_bits(int(s0[1]), int(s1[1]), L * SAMPLE_K)
        idx = (lower % np.uint32(L)).astype(np.int32).reshape(L, SAMPLE_K)
        cnt = np.zeros((L, L), dtype=np.float32)
        np.add.at(cnt, (np.arange(L)[:, None], idx), 1.0)
        cfT = np.ascontiguousarray(cnt.T)
        nmT = np.where(cfT > 0, np.float32(0.0), np.float32(NEG)).astype(np.float32)
        outs.append((nmT, cfT))
    return outs


_PE = _pos_encoding()
_CNTS = _sample_counts()


def _embed_kernel(x_ref, w_ref, b_ref, pe_ref, o_ref):
    acc = jnp.dot(x_ref[...], w_ref[...], preferred_element_type=jnp.float32)
    o_ref[...] = acc + b_ref[...] + pe_ref[...]


def _embed(xcat, w2d, bias, pe):
    blk = 512
    grid = (ROWS // blk,)
    return pl.pallas_call(
        _embed_kernel,
        grid=grid,
        in_specs=[
            pl.BlockSpec((blk, ENC_IN * 3), lambda i: (i, 0)),
            pl.BlockSpec((ENC_IN * 3, D_MODEL), lambda i: (0, 0)),
            pl.BlockSpec((1, D_MODEL), lambda i: (0, 0)),
            pl.BlockSpec((blk, D_MODEL), lambda i: (i % (L // blk), 0)),
        ],
        out_specs=pl.BlockSpec((blk, D_MODEL), lambda i: (i, 0)),
        out_shape=jax.ShapeDtypeStruct((ROWS, D_MODEL), jnp.float32),
    )(xcat, w2d, bias, pe)


def _qkv_kernel(a_ref, wq_ref, wk_ref, wv_ref, b_ref, o_ref):
    a = a_ref[...]
    b = b_ref[...]
    for j, w_ref in enumerate((wq_ref, wk_ref, wv_ref)):
        o_ref[:, j * D_MODEL:(j + 1) * D_MODEL] = (
            lax.dot_general(a, w_ref[...], (((1,), (1,)), ((), ())),
                            preferred_element_type=jnp.float32)
            + b[:, j * D_MODEL:(j + 1) * D_MODEL])


def _qkv(a, wq, wk, wv, b):
    blk = 512
    grid = (ROWS // blk,)
    wspec = pl.BlockSpec((D_MODEL, D_MODEL), lambda i: (0, 0))
    return pl.pallas_call(
        _qkv_kernel,
        grid=grid,
        in_specs=[
            pl.BlockSpec((blk, D_MODEL), lambda i: (i, 0)),
            wspec, wspec, wspec,
            pl.BlockSpec((1, 3 * D_MODEL), lambda i: (0, 0)),
        ],
        out_specs=pl.BlockSpec((blk, 3 * D_MODEL), lambda i: (i, 0)),
        out_shape=jax.ShapeDtypeStruct((ROWS, 3 * D_MODEL), jnp.float32),
    )(a, wq, wk, wv, b)


def _attn_head_body(q, k, v, nm_ref, ssum):
    chunk = 512
    m_parts = []
    for ci in range(L // chunk):
        qc = q[ci * chunk:(ci + 1) * chunk]
        st = lax.dot_general(k, qc, (((1,), (1,)), ((), ())),
                             preferred_element_type=jnp.float32)
        w = st + nm_ref[:, ci * chunk:(ci + 1) * chunk]
        m_parts.append(jnp.max(w, axis=0, keepdims=True))
    m = jnp.concatenate(m_parts, axis=1) - ssum * (1.0 / L)

    ub = lax.bitcast_convert_type(m, jnp.int32)
    si = ub ^ (lax.shift_right_arithmetic(ub, 31) & jnp.int32(0x7FFFFFFF))

    def count_ge(t):
        return jnp.sum(jnp.where(si >= t, 1.0, 0.0), axis=1, keepdims=True)

    c0 = count_ge(jnp.zeros((1, 1), jnp.int32))
    t = jnp.where(c0 >= U, jnp.int32(0), jnp.int32(-2147483648)
                  ).reshape(1, 1).astype(jnp.int32)
    for b in range(30, -1, -1):
        t_try = t + jnp.int32(1 << b)
        t = jnp.where(count_ge(t_try) >= U, t_try, t)

    mask_gt = (si > t).astype(jnp.float32)
    mask_eq = (si == t).astype(jnp.float32)
    r = U - jnp.sum(mask_gt, axis=1, keepdims=True)
    lt128 = jnp.where(
        lax.broadcasted_iota(jnp.int32, (128, 128), 0)
        <= lax.broadcasted_iota(jnp.int32, (128, 128), 1),
        1.0, 0.0).astype(jnp.bfloat16)
    lt16x = jnp.where(
        lax.broadcasted_iota(jnp.int32, (16, 16), 1)
        < lax.broadcasted_iota(jnp.int32, (16, 16), 0),
        1.0, 0.0)

    def prefix(x_row):
        xb = x_row.reshape(16, 128)
        pb = jnp.dot(xb.astype(jnp.bfloat16), lt128,
                     preferred_element_type=jnp.float32)
        off = jnp.dot(lt16x, pb[:, 127:128],
                      preferred_element_type=jnp.float32)
        return (pb + off).reshape(1, L)

    pos_eq = prefix(mask_eq)
    mask = mask_gt + mask_eq * jnp.where(pos_eq <= r, 1.0, 0.0)
    pos = prefix(mask)

    rows = lax.broadcasted_iota(jnp.int32, (U, L), 0).astype(jnp.float32) + 1.0
    oh = jnp.where((pos == rows) & (mask > 0.0), 1.0, 0.0)
    qr = jnp.dot(oh, q, preferred_element_type=jnp.float32)
    scores = lax.dot_general(qr, k, (((1,), (1,)), ((), ())),
                             preferred_element_type=jnp.float32)
    scores = scores * (1.0 / math.sqrt(DH))
    smax = jnp.max(scores, axis=1, keepdims=True)
    e = jnp.exp(scores - smax)
    attnw = e / jnp.sum(e, axis=1, keepdims=True)
    upd = jnp.dot(attnw, v, preferred_element_type=jnp.float32)

    meanv = jnp.sum(v, axis=0, keepdims=True) * (1.0 / L)
    scat = lax.dot_general(oh, upd, (((0,), (0,)), ((), ())),
                           preferred_element_type=jnp.float32)
    sel = lax.dot_general(oh, jnp.ones((U, DH), jnp.float32),
                          (((0,), (0,)), ((), ())),
                          preferred_element_type=jnp.float32)
    return scat + meanv * (1.0 - sel)


def _attn_kernel(qp_ref, kp_ref, vp_ref, nm_ref, cf_ref, o_ref):
    qp = qp_ref[0]
    kp = kp_ref[0]
    vp = vp_ref[0]
    ck2 = lax.dot_general(kp, cf_ref[...], (((0,), (0,)), ((), ())),
                          preferred_element_type=jnp.float32)
    sprod = qp.T * ck2
    ctxs = []
    for hh in range(2):
        q = qp[:, hh * DH:(hh + 1) * DH]
        k = kp[:, hh * DH:(hh + 1) * DH]
        v = vp[:, hh * DH:(hh + 1) * DH]
        ssum = jnp.sum(sprod[hh * DH:(hh + 1) * DH], axis=0, keepdims=True)
        ctxs.append(_attn_head_body(q, k, v, nm_ref, ssum))
    o_ref[0] = jnp.concatenate(ctxs, axis=1)


def _attention(qkv3, nmT, cfT):
    npair = N_HEADS // 2
    grid = (B, npair)
    qs = pl.BlockSpec((1, L, 2 * DH), lambda b, p: (b, 0, p))
    ks = pl.BlockSpec((1, L, 2 * DH), lambda b, p: (b, 0, npair + p))
    vs = pl.BlockSpec((1, L, 2 * DH), lambda b, p: (b, 0, 2 * npair + p))
    full = pl.BlockSpec((L, L), lambda b, p: (0, 0))
    return pl.pallas_call(
        _attn_kernel,
        grid=grid,
        in_specs=[qs, ks, vs, full, full],
        out_specs=pl.BlockSpec((1, L, 2 * DH), lambda b, p: (b, 0, p)),
        out_shape=jax.ShapeDtypeStruct((B, L, D_MODEL), jnp.float32),
    )(qkv3, qkv3, qkv3, nmT, cfT)


def _ln(x, g, b):
    mu = jnp.mean(x, axis=1, keepdims=True)
    xc = x - mu
    var = jnp.mean(xc * xc, axis=1, keepdims=True)
    return xc * lax.rsqrt(var + 1e-5) * g + b


def _proj_ln_kernel(a_ref, w_ref, b_ref, h_ref, g_ref, gb_ref, o_ref):
    out = (lax.dot_general(a_ref[...], w_ref[...], (((1,), (1,)), ((), ())),
                           preferred_element_type=jnp.float32)
           + b_ref[...])
    o_ref[...] = _ln(h_ref[...] + out, g_ref[...], gb_ref[...])


def _proj_ln(a, w, b, h, g, gb):
    blk = 512
    grid = (ROWS // blk,)
    return pl.pallas_call(
        _proj_ln_kernel,
        grid=grid,
        in_specs=[
            pl.BlockSpec((blk, D_MODEL), lambda i: (i, 0)),
            pl.BlockSpec((D_MODEL, D_MODEL), lambda i: (0, 0)),
            pl.BlockSpec((1, D_MODEL), lambda i: (0, 0)),
            pl.BlockSpec((blk, D_MODEL), lambda i: (i, 0)),
            pl.BlockSpec((1, D_MODEL), lambda i: (0, 0)),
            pl.BlockSpec((1, D_MODEL), lambda i: (0, 0)),
        ],
        out_specs=pl.BlockSpec((blk, D_MODEL), lambda i: (i, 0)),
        out_shape=jax.ShapeDtypeStruct((ROWS, D_MODEL), jnp.float32),
    )(a, w, b, h, g, gb)


def _ffn_kernel(h_ref, w1_ref, b1_ref, w2_ref, b2_ref, g_ref, gb_ref,
                fg_ref, fb_ref, o_ref, *, final_ln):
    h = h_ref[...]
    y = jnp.maximum(
        lax.dot_general(h, w1_ref[...], (((1,), (1,)), ((), ())),
                        preferred_element_type=jnp.float32)
        + b1_ref[...], 0.0)
    y = (lax.dot_general(y, w2_ref[...], (((1,), (1,)), ((), ())),
                         preferred_element_type=jnp.float32)
         + b2_ref[...])
    out = _ln(h + y, g_ref[...], gb_ref[...])
    if final_ln:
        out = _ln(out, fg_ref[...], fb_ref[...])
    o_ref[...] = out


def _ffn(h, w1, b1, w2, b2, g, gb, fg, fb, final_ln):
    blk = 256
    grid = (ROWS // blk,)
    vec = lambda n: pl.BlockSpec((1, n), lambda i: (0, 0))
    return pl.pallas_call(
        functools.partial(_ffn_kernel, final_ln=final_ln),
        grid=grid,
        in_specs=[
            pl.BlockSpec((blk, D_MODEL), lambda i: (i, 0)),
            pl.BlockSpec((D_FF, D_MODEL), lambda i: (0, 0)),
            vec(D_FF),
            pl.BlockSpec((D_MODEL, D_FF), lambda i: (0, 0)),
            vec(D_MODEL),
            vec(D_MODEL),
            vec(D_MODEL),
            vec(D_MODEL),
            vec(D_MODEL),
        ],
        out_specs=pl.BlockSpec((blk, D_MODEL), lambda i: (i, 0)),
        out_shape=jax.ShapeDtypeStruct((ROWS, D_MODEL), jnp.float32),
    )(h, w1, b1, w2, b2, g, gb, fg, fb)


def kernel(x, conv_w, conv_b, Wq, bq, Wk, bk, Wv, bv, Wo, bo, ln1_g, ln1_b,
           ffn1_w, ffn1_b, ffn2_w, ffn2_b, ln2_g, ln2_b, lnf_g, lnf_b):
    cnts = _CNTS
    pe = jnp.asarray(_PE)

    xprev = jnp.roll(x, 1, axis=1)
    xnext = jnp.roll(x, -1, axis=1)
    xcat = jnp.concatenate([xprev, x, xnext], axis=-1).reshape(ROWS, 3 * ENC_IN)
    w2d = conv_w.transpose(2, 1, 0).reshape(3 * ENC_IN, D_MODEL)
    h = _embed(xcat, w2d, conv_b.reshape(1, D_MODEL), pe)

    for i in range(E_LAYERS):
        bcat = jnp.concatenate([bq[i], bk[i], bv[i]]).reshape(1, 3 * D_MODEL)
        qkv = _qkv(h, Wq[i], Wk[i], Wv[i], bcat)
        ctx = _attention(qkv.reshape(B, L, 3 * D_MODEL),
                         jnp.asarray(cnts[i][0]), jnp.asarray(cnts[i][1]))
        ctx2 = ctx.reshape(ROWS, D_MODEL)
        h = _proj_ln(ctx2, Wo[i], bo[i].reshape(1, D_MODEL), h,
                     ln1_g[i].reshape(1, D_MODEL), ln1_b[i].reshape(1, D_MODEL))
        h = _ffn(h, ffn1_w[i], ffn1_b[i].reshape(1, D_FF), ffn2_w[i],
                 ffn2_b[i].reshape(1, D_MODEL),
                 ln2_g[i].reshape(1, D_MODEL), ln2_b[i].reshape(1, D_MODEL),
                 lnf_g.reshape(1, D_MODEL), lnf_b.reshape(1, D_MODEL),
                 final_ln=(i == E_LAYERS - 1))

    return h.reshape(B, L, D_MODEL)


# speedup vs baseline: 8.8808x; 1.0182x over previous
import functools
import math

import numpy as np
import jax
import jax.numpy as jnp
from jax import lax
from jax.experimental import pallas as pl
from jax.experimental.pallas import tpu as pltpu

B, L, ENC_IN = 2, 2048, 7
D_MODEL, N_HEADS, E_LAYERS, D_FF = 768, 12, 2, 3072
FACTOR = 5
DH = D_MODEL // N_HEADS
U = int(min(FACTOR * math.ceil(math.log(L)), L))
SAMPLE_K = int(min(FACTOR * math.ceil(math.log(L)), L))
ROWS = B * L
NEG = -3e38


def _pos_encoding():
    position = np.arange(L, dtype=np.float32)[:, None]
    div_term = np.exp(
        np.arange(0, D_MODEL, 2, dtype=np.float32) * (-math.log(10000.0) / D_MODEL))
    pe = np.zeros((L, D_MODEL), dtype=np.float32)
    pe[:, 0::2] = np.sin(position * div_term)
    pe[:, 1::2] = np.cos(position * div_term)
    return pe


def _threefry2x32(k0, k1, x0, x1):
    def rotl(x, d):
        return ((x << np.uint32(d)) | (x >> np.uint32(32 - d))).astype(np.uint32)
    ks0 = np.uint32(k0)
    ks1 = np.uint32(k1)
    ks2 = np.uint32(ks0 ^ ks1 ^ np.uint32(0x1BD11BDA))
    ks = [ks0, ks1, ks2]
    rot = [[13, 15, 26, 6], [17, 29, 16, 24]]
    x0 = (np.asarray(x0, np.uint32) + ks0).astype(np.uint32)
    x1 = (np.asarray(x1, np.uint32) + ks1).astype(np.uint32)
    for i in range(5):
        for r in rot[i % 2]:
            x0 = (x0 + x1).astype(np.uint32)
            x1 = rotl(x1, r)
            x1 = (x1 ^ x0).astype(np.uint32)
        x0 = (x0 + ks[(i + 1) % 3]).astype(np.uint32)
        x1 = (x1 + ks[(i + 2) % 3] + np.uint32(i + 1)).astype(np.uint32)
    return x0, x1


def _np_bits(k0, k1, size):
    cnt = np.arange(size, dtype=np.uint64)
    hi = (cnt >> np.uint64(32)).astype(np.uint32)
    lo = (cnt & np.uint64(0xFFFFFFFF)).astype(np.uint32)
    o0, o1 = _threefry2x32(k0, k1, hi, lo)
    return o0 ^ o1


@functools.lru_cache(maxsize=None)
def _sample_counts():
    outs = []
    for i in range(E_LAYERS):
        f0, f1 = _threefry2x32(0, 42, np.uint32(0), np.uint32(i))
        s0, s1 = _threefry2x32(int(f0), int(f1),
                               np.zeros(2, np.uint32),
                               np.arange(2, dtype=np.uint32))
        lower = _np_bits(int(s0[1]), int(s1[1]), L * SAMPLE_K)
        idx = (lower % np.uint32(L)).astype(np.int32).reshape(L, SAMPLE_K)
        cnt = np.zeros((L, L), dtype=np.float32)
        np.add.at(cnt, (np.arange(L)[:, None], idx), 1.0)
        cfT = np.ascontiguousarray(cnt.T)
        nmT = np.where(cfT > 0, np.float32(0.0), np.float32(NEG)).astype(np.float32)
        outs.append((nmT, cfT))
    return outs


_PE = _pos_encoding()
_CNTS = _sample_counts()


def _embed_kernel(x_ref, w_ref, b_ref, pe_ref, o_ref):
    acc = jnp.dot(x_ref[...], w_ref[...], preferred_element_type=jnp.float32)
    o_ref[...] = acc + b_ref[...] + pe_ref[...]


def _embed(xcat, w2d, bias, pe):
    blk = 512
    grid = (ROWS // blk,)
    return pl.pallas_call(
        _embed_kernel,
        grid=grid,
        in_specs=[
            pl.BlockSpec((blk, ENC_IN * 3), lambda i: (i, 0)),
            pl.BlockSpec((ENC_IN * 3, D_MODEL), lambda i: (0, 0)),
            pl.BlockSpec((1, D_MODEL), lambda i: (0, 0)),
            pl.BlockSpec((blk, D_MODEL), lambda i: (i % (L // blk), 0)),
        ],
        out_specs=pl.BlockSpec((blk, D_MODEL), lambda i: (i, 0)),
        out_shape=jax.ShapeDtypeStruct((ROWS, D_MODEL), jnp.float32),
    )(xcat, w2d, bias, pe)


def _qkv_kernel(a_ref, wq_ref, wk_ref, wv_ref, b_ref, o_ref):
    a = a_ref[...]
    b = b_ref[...]
    for j, w_ref in enumerate((wq_ref, wk_ref, wv_ref)):
        o_ref[:, j * D_MODEL:(j + 1) * D_MODEL] = (
            lax.dot_general(a, w_ref[...], (((1,), (1,)), ((), ())),
                            preferred_element_type=jnp.float32)
            + b[:, j * D_MODEL:(j + 1) * D_MODEL])


def _qkv(a, wq, wk, wv, b):
    blk = 512
    grid = (ROWS // blk,)
    wspec = pl.BlockSpec((D_MODEL, D_MODEL), lambda i: (0, 0))
    return pl.pallas_call(
        _qkv_kernel,
        grid=grid,
        in_specs=[
            pl.BlockSpec((blk, D_MODEL), lambda i: (i, 0)),
            wspec, wspec, wspec,
            pl.BlockSpec((1, 3 * D_MODEL), lambda i: (0, 0)),
        ],
        out_specs=pl.BlockSpec((blk, 3 * D_MODEL), lambda i: (i, 0)),
        out_shape=jax.ShapeDtypeStruct((ROWS, 3 * D_MODEL), jnp.float32),
    )(a, wq, wk, wv, b)


def _attn_head_body(q, k, v, nm_ref, ssum):
    chunk = 512
    m_parts = []
    for ci in range(L // chunk):
        qc = q[ci * chunk:(ci + 1) * chunk]
        st = lax.dot_general(k, qc, (((1,), (1,)), ((), ())),
                             preferred_element_type=jnp.float32)
        w = st + nm_ref[:, ci * chunk:(ci + 1) * chunk]
        m_parts.append(jnp.max(w, axis=0, keepdims=True))
    m = jnp.concatenate(m_parts, axis=1) - ssum * (1.0 / L)

    ub = lax.bitcast_convert_type(m, jnp.int32)
    si = ub ^ (lax.shift_right_arithmetic(ub, 31) & jnp.int32(0x7FFFFFFF))

    def count_ge(t):
        return jnp.sum(jnp.where(si >= t, 1.0, 0.0), axis=1, keepdims=True)

    c0 = count_ge(jnp.zeros((1, 1), jnp.int32))
    t = jnp.where(c0 >= U, jnp.int32(0), jnp.int32(-2147483648)
                  ).reshape(1, 1).astype(jnp.int32)
    for b in range(30, -1, -1):
        t_try = t + jnp.int32(1 << b)
        t = jnp.where(count_ge(t_try) >= U, t_try, t)

    mask_gt = (si > t).astype(jnp.float32)
    mask_eq = (si == t).astype(jnp.float32)
    r = U - jnp.sum(mask_gt, axis=1, keepdims=True)
    lt128 = jnp.where(
        lax.broadcasted_iota(jnp.int32, (128, 128), 0)
        <= lax.broadcasted_iota(jnp.int32, (128, 128), 1),
        1.0, 0.0).astype(jnp.bfloat16)
    lt16x = jnp.where(
        lax.broadcasted_iota(jnp.int32, (16, 16), 1)
        < lax.broadcasted_iota(jnp.int32, (16, 16), 0),
        1.0, 0.0)

    def prefix(x_row):
        xb = x_row.reshape(16, 128)
        pb = jnp.dot(xb.astype(jnp.bfloat16), lt128,
                     preferred_element_type=jnp.float32)
        off = jnp.dot(lt16x, pb[:, 127:128],
                      preferred_element_type=jnp.float32)
        return (pb + off).reshape(1, L)

    pos_eq = prefix(mask_eq)
    mask = mask_gt + mask_eq * jnp.where(pos_eq <= r, 1.0, 0.0)
    pos = prefix(mask)

    rows = lax.broadcasted_iota(jnp.int32, (U, L), 0).astype(jnp.float32) + 1.0
    oh = jnp.where((pos == rows) & (mask > 0.0), 1.0, 0.0)
    qr = jnp.dot(oh, q, preferred_element_type=jnp.float32)
    scores = lax.dot_general(qr, k, (((1,), (1,)), ((), ())),
                             preferred_element_type=jnp.float32)
    scores = scores * (1.0 / math.sqrt(DH))
    smax = jnp.max(scores, axis=1, keepdims=True)
    e = jnp.exp(scores - smax)
    attnw = e / jnp.sum(e, axis=1, keepdims=True)
    upd = jnp.dot(attnw, v, preferred_element_type=jnp.float32)

    meanv = jnp.sum(v, axis=0, keepdims=True) * (1.0 / L)
    scat = lax.dot_general(oh, upd, (((0,), (0,)), ((), ())),
                           preferred_element_type=jnp.float32)
    sel = lax.dot_general(oh, jnp.ones((U, DH), jnp.float32),
                          (((0,), (0,)), ((), ())),
                          preferred_element_type=jnp.float32)
    return scat + meanv * (1.0 - sel)


HPG = 4


def _attn_kernel(qp_ref, kp_ref, vp_ref, nm_ref, cf_ref, o_ref):
    qp = qp_ref[0]
    kp = kp_ref[0]
    vp = vp_ref[0]
    ckg = lax.dot_general(kp, cf_ref[...], (((0,), (0,)), ((), ())),
                          preferred_element_type=jnp.float32)
    sprod = qp.T * ckg
    ctxs = []
    for hh in range(HPG):
        q = qp[:, hh * DH:(hh + 1) * DH]
        k = kp[:, hh * DH:(hh + 1) * DH]
        v = vp[:, hh * DH:(hh + 1) * DH]
        ssum = jnp.sum(sprod[hh * DH:(hh + 1) * DH], axis=0, keepdims=True)
        ctxs.append(_attn_head_body(q, k, v, nm_ref, ssum))
    o_ref[0] = jnp.concatenate(ctxs, axis=1)


def _attention(qkv3, nmT, cfT):
    ng = N_HEADS // HPG
    grid = (B, ng)
    qs = pl.BlockSpec((1, L, HPG * DH), lambda b, p: (b, 0, p))
    ks = pl.BlockSpec((1, L, HPG * DH), lambda b, p: (b, 0, ng + p))
    vs = pl.BlockSpec((1, L, HPG * DH), lambda b, p: (b, 0, 2 * ng + p))
    full = pl.BlockSpec((L, L), lambda b, p: (0, 0))
    return pl.pallas_call(
        _attn_kernel,
        grid=grid,
        in_specs=[qs, ks, vs, full, full],
        out_specs=pl.BlockSpec((1, L, HPG * DH), lambda b, p: (b, 0, p)),
        out_shape=jax.ShapeDtypeStruct((B, L, D_MODEL), jnp.float32),
    )(qkv3, qkv3, qkv3, nmT, cfT)


def _ln(x, g, b):
    mu = jnp.mean(x, axis=1, keepdims=True)
    xc = x - mu
    var = jnp.mean(xc * xc, axis=1, keepdims=True)
    return xc * lax.rsqrt(var + 1e-5) * g + b


def _proj_ln_kernel(a_ref, w_ref, b_ref, h_ref, g_ref, gb_ref, o_ref):
    out = (lax.dot_general(a_ref[...], w_ref[...], (((1,), (1,)), ((), ())),
                           preferred_element_type=jnp.float32)
           + b_ref[...])
    o_ref[...] = _ln(h_ref[...] + out, g_ref[...], gb_ref[...])


def _proj_ln(a, w, b, h, g, gb):
    blk = 512
    grid = (ROWS // blk,)
    return pl.pallas_call(
        _proj_ln_kernel,
        grid=grid,
        in_specs=[
            pl.BlockSpec((blk, D_MODEL), lambda i: (i, 0)),
            pl.BlockSpec((D_MODEL, D_MODEL), lambda i: (0, 0)),
            pl.BlockSpec((1, D_MODEL), lambda i: (0, 0)),
            pl.BlockSpec((blk, D_MODEL), lambda i: (i, 0)),
            pl.BlockSpec((1, D_MODEL), lambda i: (0, 0)),
            pl.BlockSpec((1, D_MODEL), lambda i: (0, 0)),
        ],
        out_specs=pl.BlockSpec((blk, D_MODEL), lambda i: (i, 0)),
        out_shape=jax.ShapeDtypeStruct((ROWS, D_MODEL), jnp.float32),
    )(a, w, b, h, g, gb)


def _ffn_kernel(h_ref, w1_ref, b1_ref, w2_ref, b2_ref, g_ref, gb_ref,
                fg_ref, fb_ref, o_ref, *, final_ln):
    h = h_ref[...]
    y = jnp.maximum(
        lax.dot_general(h, w1_ref[...], (((1,), (1,)), ((), ())),
                        preferred_element_type=jnp.float32)
        + b1_ref[...], 0.0)
    y = (lax.dot_general(y, w2_ref[...], (((1,), (1,)), ((), ())),
                         preferred_element_type=jnp.float32)
         + b2_ref[...])
    out = _ln(h + y, g_ref[...], gb_ref[...])
    if final_ln:
        out = _ln(out, fg_ref[...], fb_ref[...])
    o_ref[...] = out


def _ffn(h, w1, b1, w2, b2, g, gb, fg, fb, final_ln):
    blk = 256
    grid = (ROWS // blk,)
    vec = lambda n: pl.BlockSpec((1, n), lambda i: (0, 0))
    return pl.pallas_call(
        functools.partial(_ffn_kernel, final_ln=final_ln),
        grid=grid,
        in_specs=[
            pl.BlockSpec((blk, D_MODEL), lambda i: (i, 0)),
            pl.BlockSpec((D_FF, D_MODEL), lambda i: (0, 0)),
            vec(D_FF),
            pl.BlockSpec((D_MODEL, D_FF), lambda i: (0, 0)),
            vec(D_MODEL),
            vec(D_MODEL),
            vec(D_MODEL),
            vec(D_MODEL),
            vec(D_MODEL),
        ],
        out_specs=pl.BlockSpec((blk, D_MODEL), lambda i: (i, 0)),
        out_shape=jax.ShapeDtypeStruct((ROWS, D_MODEL), jnp.float32),
    )(h, w1, b1, w2, b2, g, gb, fg, fb)


def kernel(x, conv_w, conv_b, Wq, bq, Wk, bk, Wv, bv, Wo, bo, ln1_g, ln1_b,
           ffn1_w, ffn1_b, ffn2_w, ffn2_b, ln2_g, ln2_b, lnf_g, lnf_b):
    cnts = _CNTS
    pe = jnp.asarray(_PE)

    xprev = jnp.roll(x, 1, axis=1)
    xnext = jnp.roll(x, -1, axis=1)
    xcat = jnp.concatenate([xprev, x, xnext], axis=-1).reshape(ROWS, 3 * ENC_IN)
    w2d = conv_w.transpose(2, 1, 0).reshape(3 * ENC_IN, D_MODEL)
    h = _embed(xcat, w2d, conv_b.reshape(1, D_MODEL), pe)

    for i in range(E_LAYERS):
        bcat = jnp.concatenate([bq[i], bk[i], bv[i]]).reshape(1, 3 * D_MODEL)
        qkv = _qkv(h, Wq[i], Wk[i], Wv[i], bcat)
        ctx = _attention(qkv.reshape(B, L, 3 * D_MODEL),
                         jnp.asarray(cnts[i][0]), jnp.asarray(cnts[i][1]))
        ctx2 = ctx.reshape(ROWS, D_MODEL)
        h = _proj_ln(ctx2, Wo[i], bo[i].reshape(1, D_MODEL), h,
                     ln1_g[i].reshape(1, D_MODEL), ln1_b[i].reshape(1, D_MODEL))
        h = _ffn(h, ffn1_w[i], ffn1_b[i].reshape(1, D_FF), ffn2_w[i],
                 ffn2_b[i].reshape(1, D_MODEL),
                 ln2_g[i].reshape(1, D_MODEL), ln2_b[i].reshape(1, D_MODEL),
                 lnf_g.reshape(1, D_MODEL), lnf_b.reshape(1, D_MODEL),
                 final_ln=(i == E_LAYERS - 1))

    return h.reshape(B, L, D_MODEL)


# speedup vs baseline: 9.0111x; 1.0147x over previous
import functools
import math

import numpy as np
import jax
import jax.numpy as jnp
from jax import lax
from jax.experimental import pallas as pl
from jax.experimental.pallas import tpu as pltpu

B, L, ENC_IN = 2, 2048, 7
D_MODEL, N_HEADS, E_LAYERS, D_FF = 768, 12, 2, 3072
FACTOR = 5
DH = D_MODEL // N_HEADS
U = int(min(FACTOR * math.ceil(math.log(L)), L))
SAMPLE_K = int(min(FACTOR * math.ceil(math.log(L)), L))
ROWS = B * L
NEG = -3e38


def _pos_encoding():
    position = np.arange(L, dtype=np.float32)[:, None]
    div_term = np.exp(
        np.arange(0, D_MODEL, 2, dtype=np.float32) * (-math.log(10000.0) / D_MODEL))
    pe = np.zeros((L, D_MODEL), dtype=np.float32)
    pe[:, 0::2] = np.sin(position * div_term)
    pe[:, 1::2] = np.cos(position * div_term)
    return pe


def _threefry2x32(k0, k1, x0, x1):
    def rotl(x, d):
        return ((x << np.uint32(d)) | (x >> np.uint32(32 - d))).astype(np.uint32)
    ks0 = np.uint32(k0)
    ks1 = np.uint32(k1)
    ks2 = np.uint32(ks0 ^ ks1 ^ np.uint32(0x1BD11BDA))
    ks = [ks0, ks1, ks2]
    rot = [[13, 15, 26, 6], [17, 29, 16, 24]]
    x0 = (np.asarray(x0, np.uint32) + ks0).astype(np.uint32)
    x1 = (np.asarray(x1, np.uint32) + ks1).astype(np.uint32)
    for i in range(5):
        for r in rot[i % 2]:
            x0 = (x0 + x1).astype(np.uint32)
            x1 = rotl(x1, r)
            x1 = (x1 ^ x0).astype(np.uint32)
        x0 = (x0 + ks[(i + 1) % 3]).astype(np.uint32)
        x1 = (x1 + ks[(i + 2) % 3] + np.uint32(i + 1)).astype(np.uint32)
    return x0, x1


def _np_bits(k0, k1, size):
    cnt = np.arange(size, dtype=np.uint64)
    hi = (cnt >> np.uint64(32)).astype(np.uint32)
    lo = (cnt & np.uint64(0xFFFFFFFF)).astype(np.uint32)
    o0, o1 = _threefry2x32(k0, k1, hi, lo)
    return o0 ^ o1


@functools.lru_cache(maxsize=None)
def _sample_counts():
    outs = []
    for i in range(E_LAYERS):
        f0, f1 = _threefry2x32(0, 42, np.uint32(0), np.uint32(i))
        s0, s1 = _threefry2x32(int(f0), int(f1),
                               np.zeros(2, np.uint32),
                               np.arange(2, dtype=np.uint32))
        lower = _np_bits(int(s0[1]), int(s1[1]), L * SAMPLE_K)
        idx = (lower % np.uint32(L)).astype(np.int32).reshape(L, SAMPLE_K)
        cnt = np.zeros((L, L), dtype=np.float32)
        np.add.at(cnt, (np.arange(L)[:, None], idx), 1.0)
        cfT = np.ascontiguousarray(cnt.T)
        nmT = np.where(cfT > 0, np.float32(0.0), np.float32(NEG)).astype(np.float32)
        outs.append((nmT, cfT))
    return outs


_PE = _pos_encoding()
_CNTS = _sample_counts()


def _embed_kernel(x_ref, w_ref, b_ref, pe_ref, o_ref):
    acc = jnp.dot(x_ref[...], w_ref[...], preferred_element_type=jnp.float32)
    o_ref[...] = acc + b_ref[...] + pe_ref[...]


def _embed(xcat, w2d, bias, pe):
    blk = 512
    grid = (ROWS // blk,)
    return pl.pallas_call(
        _embed_kernel,
        grid=grid,
        in_specs=[
            pl.BlockSpec((blk, ENC_IN * 3), lambda i: (i, 0)),
            pl.BlockSpec((ENC_IN * 3, D_MODEL), lambda i: (0, 0)),
            pl.BlockSpec((1, D_MODEL), lambda i: (0, 0)),
            pl.BlockSpec((blk, D_MODEL), lambda i: (i % (L // blk), 0)),
        ],
        out_specs=pl.BlockSpec((blk, D_MODEL), lambda i: (i, 0)),
        out_shape=jax.ShapeDtypeStruct((ROWS, D_MODEL), jnp.float32),
    )(xcat, w2d, bias, pe)


def _qkv_kernel(a_ref, wq_ref, wk_ref, wv_ref, b_ref, o_ref):
    a = a_ref[...]
    b = b_ref[...]
    for j, w_ref in enumerate((wq_ref, wk_ref, wv_ref)):
        o_ref[:, j * D_MODEL:(j + 1) * D_MODEL] = (
            lax.dot_general(a, w_ref[...], (((1,), (1,)), ((), ())),
                            preferred_element_type=jnp.float32)
            + b[:, j * D_MODEL:(j + 1) * D_MODEL])


def _qkv(a, wq, wk, wv, b):
    blk = 512
    grid = (ROWS // blk,)
    wspec = pl.BlockSpec((D_MODEL, D_MODEL), lambda i: (0, 0))
    return pl.pallas_call(
        _qkv_kernel,
        grid=grid,
        in_specs=[
            pl.BlockSpec((blk, D_MODEL), lambda i: (i, 0)),
            wspec, wspec, wspec,
            pl.BlockSpec((1, 3 * D_MODEL), lambda i: (0, 0)),
        ],
        out_specs=pl.BlockSpec((blk, 3 * D_MODEL), lambda i: (i, 0)),
        out_shape=jax.ShapeDtypeStruct((ROWS, 3 * D_MODEL), jnp.float32),
    )(a, wq, wk, wv, b)


def _attn_head_body(q, k, v, nm_ref, ssum):
    chunk = 512
    m_parts = []
    for ci in range(L // chunk):
        qc = q[ci * chunk:(ci + 1) * chunk]
        st = lax.dot_general(k, qc, (((1,), (1,)), ((), ())),
                             preferred_element_type=jnp.float32)
        w = st + nm_ref[:, ci * chunk:(ci + 1) * chunk]
        m_parts.append(jnp.max(w, axis=0, keepdims=True))
    m = jnp.concatenate(m_parts, axis=1) - ssum * (1.0 / L)

    ub = lax.bitcast_convert_type(m, jnp.int32)
    si = ub ^ (lax.shift_right_arithmetic(ub, 31) & jnp.int32(0x7FFFFFFF))

    def count_ge(t):
        return jnp.sum(jnp.where(si >= t, 1.0, 0.0), axis=1, keepdims=True)

    c0 = count_ge(jnp.zeros((1, 1), jnp.int32))
    t = jnp.where(c0 >= U, jnp.int32(0), jnp.int32(-2147483648)
                  ).reshape(1, 1).astype(jnp.int32)
    for b in range(30, -1, -1):
        t_try = t + jnp.int32(1 << b)
        t = jnp.where(count_ge(t_try) >= U, t_try, t)

    mask_gt = (si > t).astype(jnp.float32)
    mask_eq = (si == t).astype(jnp.float32)
    r = U - jnp.sum(mask_gt, axis=1, keepdims=True)
    lt128 = jnp.where(
        lax.broadcasted_iota(jnp.int32, (128, 128), 0)
        <= lax.broadcasted_iota(jnp.int32, (128, 128), 1),
        1.0, 0.0).astype(jnp.bfloat16)
    lt16x = jnp.where(
        lax.broadcasted_iota(jnp.int32, (16, 16), 1)
        < lax.broadcasted_iota(jnp.int32, (16, 16), 0),
        1.0, 0.0)

    def prefix(x_row):
        xb = x_row.reshape(16, 128)
        pb = jnp.dot(xb.astype(jnp.bfloat16), lt128,
                     preferred_element_type=jnp.float32)
        off = jnp.dot(lt16x, pb[:, 127:128],
                      preferred_element_type=jnp.float32)
        return (pb + off).reshape(1, L)

    pos_eq = prefix(mask_eq)
    mask = mask_gt + mask_eq * jnp.where(pos_eq <= r, 1.0, 0.0)
    pos = prefix(mask)

    rows = lax.broadcasted_iota(jnp.int32, (U, L), 0).astype(jnp.float32) + 1.0
    oh = jnp.where((pos == rows) & (mask > 0.0), 1.0, 0.0)
    qr = jnp.dot(oh, q, preferred_element_type=jnp.float32)
    scores = lax.dot_general(qr, k, (((1,), (1,)), ((), ())),
                             preferred_element_type=jnp.float32)
    scores = scores * (1.0 / math.sqrt(DH))
    smax = jnp.max(scores, axis=1, keepdims=True)
    e = jnp.exp(scores - smax)
    attnw = e / jnp.sum(e, axis=1, keepdims=True)
    upd = jnp.dot(attnw, v, preferred_element_type=jnp.float32)

    meanv = jnp.sum(v, axis=0, keepdims=True) * (1.0 / L)
    scat = lax.dot_general(oh, upd, (((0,), (0,)), ((), ())),
                           preferred_element_type=jnp.float32)
    sel = lax.dot_general(oh, jnp.ones((U, DH), jnp.float32),
                          (((0,), (0,)), ((), ())),
                          preferred_element_type=jnp.float32)
    return scat + meanv * (1.0 - sel)


HPG = 4


def _attn_kernel(qp_ref, kp_ref, vp_ref, nm_ref, cf_ref, o_ref):
    qp = qp_ref[0]
    kp = kp_ref[0]
    vp = vp_ref[0]
    ckg = lax.dot_general(kp, cf_ref[...], (((0,), (0,)), ((), ())),
                          preferred_element_type=jnp.float32)
    sprod = qp.T * ckg
    ctxs = []
    for hh in range(HPG):
        q = qp[:, hh * DH:(hh + 1) * DH]
        k = kp[:, hh * DH:(hh + 1) * DH]
        v = vp[:, hh * DH:(hh + 1) * DH]
        ssum = jnp.sum(sprod[hh * DH:(hh + 1) * DH], axis=0, keepdims=True)
        ctxs.append(_attn_head_body(q, k, v, nm_ref, ssum))
    o_ref[0] = jnp.concatenate(ctxs, axis=1)


def _attention(qkv3, nmT, cfT):
    ng = N_HEADS // HPG
    grid = (B, ng)
    qs = pl.BlockSpec((1, L, HPG * DH), lambda b, p: (b, 0, p))
    ks = pl.BlockSpec((1, L, HPG * DH), lambda b, p: (b, 0, ng + p))
    vs = pl.BlockSpec((1, L, HPG * DH), lambda b, p: (b, 0, 2 * ng + p))
    full = pl.BlockSpec((L, L), lambda b, p: (0, 0))
    return pl.pallas_call(
        _attn_kernel,
        grid=grid,
        in_specs=[qs, ks, vs, full, full],
        out_specs=pl.BlockSpec((1, L, HPG * DH), lambda b, p: (b, 0, p)),
        out_shape=jax.ShapeDtypeStruct((B, L, D_MODEL), jnp.float32),
    )(qkv3, qkv3, qkv3, nmT, cfT)


def _ln(x, g, b):
    mu = jnp.mean(x, axis=1, keepdims=True)
    xc = x - mu
    var = jnp.mean(xc * xc, axis=1, keepdims=True)
    return xc * lax.rsqrt(var + 1e-5) * g + b


def _block_kernel(a_ref, wo_ref, bo_ref, h_ref, g1_ref, gb1_ref,
                  w1_ref, b1_ref, w2_ref, b2_ref, g2_ref, gb2_ref,
                  fg_ref, fb_ref, o_ref, *, final_ln):
    out = (lax.dot_general(a_ref[...], wo_ref[...], (((1,), (1,)), ((), ())),
                           preferred_element_type=jnp.float32)
           + bo_ref[...])
    h1 = _ln(h_ref[...] + out, g1_ref[...], gb1_ref[...])
    y = jnp.maximum(
        lax.dot_general(h1, w1_ref[...], (((1,), (1,)), ((), ())),
                        preferred_element_type=jnp.float32)
        + b1_ref[...], 0.0)
    y = (lax.dot_general(y, w2_ref[...], (((1,), (1,)), ((), ())),
                         preferred_element_type=jnp.float32)
         + b2_ref[...])
    out2 = _ln(h1 + y, g2_ref[...], gb2_ref[...])
    if final_ln:
        out2 = _ln(out2, fg_ref[...], fb_ref[...])
    o_ref[...] = out2


def _block(a, wo, bo, h, g1, gb1, w1, b1, w2, b2, g2, gb2, fg, fb, final_ln):
    blk = 512
    grid = (ROWS // blk,)
    vec = lambda n: pl.BlockSpec((1, n), lambda i: (0, 0))
    rowspec = pl.BlockSpec((blk, D_MODEL), lambda i: (i, 0))
    return pl.pallas_call(
        functools.partial(_block_kernel, final_ln=final_ln),
        grid=grid,
        in_specs=[
            rowspec,
            pl.BlockSpec((D_MODEL, D_MODEL), lambda i: (0, 0)),
            vec(D_MODEL),
            rowspec,
            vec(D_MODEL),
            vec(D_MODEL),
            pl.BlockSpec((D_FF, D_MODEL), lambda i: (0, 0)),
            vec(D_FF),
            pl.BlockSpec((D_MODEL, D_FF), lambda i: (0, 0)),
            vec(D_MODEL),
            vec(D_MODEL),
            vec(D_MODEL),
            vec(D_MODEL),
            vec(D_MODEL),
        ],
        out_specs=rowspec,
        out_shape=jax.ShapeDtypeStruct((ROWS, D_MODEL), jnp.float32),
    )(a, wo, bo, h, g1, gb1, w1, b1, w2, b2, g2, gb2, fg, fb)


def kernel(x, conv_w, conv_b, Wq, bq, Wk, bk, Wv, bv, Wo, bo, ln1_g, ln1_b,
           ffn1_w, ffn1_b, ffn2_w, ffn2_b, ln2_g, ln2_b, lnf_g, lnf_b):
    cnts = _CNTS
    pe = jnp.asarray(_PE)

    xprev = jnp.roll(x, 1, axis=1)
    xnext = jnp.roll(x, -1, axis=1)
    xcat = jnp.concatenate([xprev, x, xnext], axis=-1).reshape(ROWS, 3 * ENC_IN)
    w2d = conv_w.transpose(2, 1, 0).reshape(3 * ENC_IN, D_MODEL)
    h = _embed(xcat, w2d, conv_b.reshape(1, D_MODEL), pe)

    for i in range(E_LAYERS):
        bcat = jnp.concatenate([bq[i], bk[i], bv[i]]).reshape(1, 3 * D_MODEL)
        qkv = _qkv(h, Wq[i], Wk[i], Wv[i], bcat)
        ctx = _attention(qkv.reshape(B, L, 3 * D_MODEL),
                         jnp.asarray(cnts[i][0]), jnp.asarray(cnts[i][1]))
        ctx2 = ctx.reshape(ROWS, D_MODEL)
        h = _block(ctx2, Wo[i], bo[i].reshape(1, D_MODEL), h,
                   ln1_g[i].reshape(1, D_MODEL), ln1_b[i].reshape(1, D_MODEL),
                   ffn1_w[i], ffn1_b[i].reshape(1, D_FF), ffn2_w[i],
                   ffn2_b[i].reshape(1, D_MODEL),
                   ln2_g[i].reshape(1, D_MODEL), ln2_b[i].reshape(1, D_MODEL),
                   lnf_g.reshape(1, D_MODEL), lnf_b.reshape(1, D_MODEL),
                   final_ln=(i == E_LAYERS - 1))

    return h.reshape(B, L, D_MODEL)
